# baseline (speedup 1.0000x reference)
import functools

import jax
import jax.numpy as jnp
from jax import lax
from jax.experimental import pallas as pl
from jax.experimental.pallas import tpu as pltpu
from jax.experimental.pallas import tpu_sc as plsc

_SC_ROWS = 3072


def _make_sc_add(R_sc, D, row0):
    info = plsc.get_sparse_core_info()
    NC, NS = info.num_cores, info.num_subcores
    NW = NC * NS
    CH = 8
    rows_per_w = R_sc // NW
    n_chunks = rows_per_w // CH
    n_groups = n_chunks // 2
    mesh = plsc.VectorSubcoreMesh(core_axis_name="c", subcore_axis_name="s")

    @functools.partial(
        pl.kernel,
        out_type=jax.ShapeDtypeStruct((R_sc, D), jnp.float32),
        mesh=mesh,
        scratch_types=[
            pltpu.VMEM((CH, D), jnp.float32),
            pltpu.VMEM((CH, D), jnp.float32),
            pltpu.VMEM((CH, D), jnp.float32),
            pltpu.VMEM((CH, D), jnp.float32),
            pltpu.SemaphoreType.DMA,
            pltpu.SemaphoreType.DMA,
            pltpu.SemaphoreType.DMA,
            pltpu.SemaphoreType.DMA,
            pltpu.SemaphoreType.DMA,
            pltpu.SemaphoreType.DMA,
        ],
    )
    def sc_add(x_hbm, e_hbm, out_hbm, bx0, be0, bx1, be1, lx0, le0, lx1,
               le1, so0, so1):
        bufx = (bx0, bx1)
        bufe = (be0, be1)
        slx = (lx0, lx1)
        sle = (le0, le1)
        sso = (so0, so1)
        wid = lax.axis_index("s") * NC + lax.axis_index("c")
        w_base = wid * rows_per_w

        def in_rows(k):
            return pl.ds(row0 + w_base + k * CH, CH)

        def out_rows(k):
            return pl.ds(w_base + k * CH, CH)

        for b in range(2):
            pltpu.async_copy(x_hbm.at[in_rows(b)], bufx[b], slx[b])
            pltpu.async_copy(e_hbm.at[in_rows(b)], bufe[b], sle[b])

        def group_body(g, carry):
            for b in range(2):
                k = g * 2 + b
                pltpu.make_async_copy(
                    x_hbm.at[in_rows(k)], bufx[b], slx[b]
                ).wait()
                pltpu.make_async_copy(
                    e_hbm.at[in_rows(k)], bufe[b], sle[b]
                ).wait()

                for r in range(CH):

                    @plsc.parallel_loop(0, D, 16, unroll=8)
                    def _body(i):
                        plsc.addupdate(
                            bufx[b].at[r, pl.ds(i, 16)],
                            bufe[b][r, pl.ds(i, 16)],
                        )

                pltpu.async_copy(bufx[b], out_hbm.at[out_rows(k)], sso[b])

                @pl.when(k + 2 < n_chunks)
                def _next():
                    pltpu.make_async_copy(
                        bufx[b], out_hbm.at[out_rows(k)], sso[b]
                    ).wait()
                    pltpu.async_copy(x_hbm.at[in_rows(k + 2)], bufx[b], slx[b])
                    pltpu.async_copy(e_hbm.at[in_rows(k + 2)], bufe[b], sle[b])

            return carry

        lax.fori_loop(0, n_groups, group_body, 0)

        for b in range(2):
            pltpu.make_async_copy(
                bufx[b], out_hbm.at[out_rows(n_chunks - 2 + b)], sso[b]
            ).wait()

    return sc_add


def _tc_add_body(x_ref, e_ref, o_ref):
    o_ref[...] = x_ref[...] + e_ref[...]


def _tc_merge_body(_full_ref, sc_ref, o_ref):
    o_ref[...] = sc_ref[...]


def kernel(x, node_pos_emb):
    B, L, D = x.shape
    R = B * L
    x2 = x.reshape(R, D)
    e2 = node_pos_emb.reshape(R, D)
    S = R - _SC_ROWS

    BLK = 256
    tc_out = pl.pallas_call(
        _tc_add_body,
        grid=(S // BLK,),
        in_specs=[
            pl.BlockSpec((BLK, D), lambda i: (i, 0)),
            pl.BlockSpec((BLK, D), lambda i: (i, 0)),
        ],
        out_specs=pl.BlockSpec((BLK, D), lambda i: (i, 0)),
        out_shape=jax.ShapeDtypeStruct((R, D), x.dtype),
    )(x2, e2)

    sc_out = _make_sc_add(_SC_ROWS, D, S)(x2, e2)

    off = S // BLK
    out = pl.pallas_call(
        _tc_merge_body,
        grid=(_SC_ROWS // BLK,),
        in_specs=[
            pl.BlockSpec(memory_space=pl.ANY),
            pl.BlockSpec((BLK, D), lambda i: (i, 0)),
        ],
        out_specs=pl.BlockSpec((BLK, D), lambda i: (i + off, 0)),
        out_shape=jax.ShapeDtypeStruct((R, D), x.dtype),
        input_output_aliases={0: 0},
    )(tc_out, sc_out)

    return out.reshape(B, L, D)


# speedup vs baseline: 1.5841x; 1.5841x over previous
import jax
import jax.numpy as jnp
from jax.experimental import pallas as pl


def _add_body(x_ref, e_ref, o_ref):
    o_ref[...] = x_ref[...] + e_ref[...]


def kernel(x, node_pos_emb):
    B, L, D = x.shape
    R = B * L
    x2 = x.reshape(R, D)
    e2 = node_pos_emb.reshape(R, D)
    BLK = 1024
    out = pl.pallas_call(
        _add_body,
        grid=(R // BLK,),
        in_specs=[
            pl.BlockSpec((BLK, D), lambda i: (i, 0)),
            pl.BlockSpec((BLK, D), lambda i: (i, 0)),
        ],
        out_specs=pl.BlockSpec((BLK, D), lambda i: (i, 0)),
        out_shape=jax.ShapeDtypeStruct((R, D), x.dtype),
    )(x2, e2)
    return out.reshape(B, L, D)
